# baseline (speedup 1.0000x reference)
import functools
import math

import jax
import jax.numpy as jnp
from jax.experimental import pallas as pl
from jax.experimental.pallas import tpu as pltpu

SEQ_LEN = 4096
D_MODEL = 2048
D_HALF = D_MODEL // 2
SEG_ID_CLS = 2
BLOCK_ROWS = 256
NUM_BLOCKS = SEQ_LEN // BLOCK_ROWS
LN10000 = math.log(10000.0)


def _attn_struct_kernel(seg_ref, q1_ref, q2_ref, k1_ref, k2_ref,
                        seg_out_ref, func_ref, sd_ref, cd_ref):
    i = pl.program_id(0)
    row0 = i * BLOCK_ROWS

    freq = jax.lax.broadcasted_iota(jnp.int32, (1, D_HALF), 1).astype(jnp.float32)
    inv_freq = jnp.exp(freq * (-LN10000 / D_HALF))

    @pl.when(i == 0)
    def _init_tables():
        dr = jax.lax.broadcasted_iota(
            jnp.int32, (BLOCK_ROWS, 1), 0).astype(jnp.float32)
        ang = dr * inv_freq
        sd_ref[...] = jnp.sin(ang)
        cd_ref[...] = jnp.cos(ang)

    base = row0.astype(jnp.float32) * inv_freq
    sb = jnp.sin(base)
    cb = jnp.cos(base)
    sd = sd_ref[...]
    cd = cd_ref[...]
    s = sb * cd + cb * sd
    c = cb * cd - sb * sd

    q1_ref[...] = jnp.concatenate([s, s], axis=-1)
    q2_ref[...] = jnp.concatenate([c, c], axis=-1)
    k1_ref[...] = jnp.concatenate([c, s], axis=-1)
    k2_ref[...] = jnp.concatenate([-s, c], axis=-1)

    rows = jax.lax.broadcasted_iota(jnp.int32, (BLOCK_ROWS, 1), 0) + row0
    col_pos = jax.lax.broadcasted_iota(jnp.int32, (BLOCK_ROWS, SEQ_LEN), 1)
    row_nz = (rows > 0).astype(jnp.float32)
    col_nz = (col_pos > 0).astype(jnp.float32)
    func_ref[...] = row_nz * col_nz

    seg_full = seg_ref[...]
    seg_rows = seg_ref[:, pl.ds(row0, BLOCK_ROWS)]
    a = seg_rows[:, :, None]
    b = seg_full[:, None, :]
    eq = (a == b) | (a == SEG_ID_CLS) | (b == SEG_ID_CLS)
    seg_out_ref[...] = eq.astype(jnp.int8)


@functools.partial(jax.jit, static_argnames=("interpret",))
def _run(seg_id, interpret=False):
    enc_shape = jax.ShapeDtypeStruct((SEQ_LEN, D_MODEL), jnp.float32)
    out_shapes = (
        enc_shape, enc_shape, enc_shape, enc_shape,
        jax.ShapeDtypeStruct((2, SEQ_LEN, SEQ_LEN), jnp.int8),
        jax.ShapeDtypeStruct((SEQ_LEN, SEQ_LEN), jnp.float32),
    )
    enc_spec = pl.BlockSpec((BLOCK_ROWS, D_MODEL), lambda i: (i, 0))
    out_specs = (
        enc_spec, enc_spec, enc_spec, enc_spec,
        pl.BlockSpec((2, BLOCK_ROWS, SEQ_LEN), lambda i: (0, i, 0)),
        pl.BlockSpec((BLOCK_ROWS, SEQ_LEN), lambda i: (i, 0)),
    )
    in_specs = [pl.BlockSpec((2, SEQ_LEN), lambda i: (0, 0))]
    return pl.pallas_call(
        _attn_struct_kernel,
        grid=(NUM_BLOCKS,),
        in_specs=in_specs,
        out_specs=out_specs,
        out_shape=out_shapes,
        scratch_shapes=[
            pltpu.VMEM((BLOCK_ROWS, D_HALF), jnp.float32),
            pltpu.VMEM((BLOCK_ROWS, D_HALF), jnp.float32),
        ],
        interpret=interpret,
    )(seg_id)


def kernel(hidden, seg_id, input_mask):
    del hidden
    q1, q2, k1, k2, seg_i8, func_mask = _run(seg_id)
    seg_mat = seg_i8.astype(jnp.bool_)
    attn_mask = input_mask[:, None, None, :]
    return (q1, q2, k1, k2, seg_mat, attn_mask, func_mask)


# speedup vs baseline: 1.0182x; 1.0182x over previous
import functools
import math

import jax
import jax.numpy as jnp
from jax.experimental import pallas as pl
from jax.experimental.pallas import tpu as pltpu

SEQ_LEN = 4096
D_MODEL = 2048
D_HALF = D_MODEL // 2
SEG_ID_CLS = 2
BLOCK_ROWS = 128
NUM_BLOCKS = SEQ_LEN // BLOCK_ROWS
LN10000 = math.log(10000.0)


def _attn_struct_kernel(seg_ref, q1_ref, q2_ref, k1_ref, k2_ref,
                        seg_out_ref, func_ref, sd_ref, cd_ref):
    i = pl.program_id(0)
    row0 = i * BLOCK_ROWS

    freq = jax.lax.broadcasted_iota(jnp.int32, (1, D_HALF), 1).astype(jnp.float32)
    inv_freq = jnp.exp(freq * (-LN10000 / D_HALF))

    @pl.when(i == 0)
    def _init_tables():
        dr = jax.lax.broadcasted_iota(
            jnp.int32, (BLOCK_ROWS, 1), 0).astype(jnp.float32)
        ang = dr * inv_freq
        sd_ref[...] = jnp.sin(ang)
        cd_ref[...] = jnp.cos(ang)

    base = row0.astype(jnp.float32) * inv_freq
    sb = jnp.sin(base)
    cb = jnp.cos(base)
    sd = sd_ref[...]
    cd = cd_ref[...]
    s = sb * cd + cb * sd
    c = cb * cd - sb * sd

    q1_ref[...] = jnp.concatenate([s, s], axis=-1)
    q2_ref[...] = jnp.concatenate([c, c], axis=-1)
    k1_ref[...] = jnp.concatenate([c, s], axis=-1)
    k2_ref[...] = jnp.concatenate([-s, c], axis=-1)

    rows = jax.lax.broadcasted_iota(jnp.int32, (BLOCK_ROWS, 1), 0) + row0
    col_pos = jax.lax.broadcasted_iota(jnp.int32, (BLOCK_ROWS, SEQ_LEN), 1)
    row_nz = (rows > 0).astype(jnp.float32)
    col_nz = (col_pos > 0).astype(jnp.float32)
    func_ref[...] = row_nz * col_nz

    seg_full = seg_ref[...]
    seg_rows = seg_ref[:, pl.ds(row0, BLOCK_ROWS)]
    a = seg_rows[:, :, None]
    b = seg_full[:, None, :]
    eq = (a == b) | (a == SEG_ID_CLS) | (b == SEG_ID_CLS)
    seg_out_ref[...] = eq.astype(jnp.int8)


@functools.partial(jax.jit, static_argnames=("interpret",))
def _run(seg_id, interpret=False):
    enc_shape = jax.ShapeDtypeStruct((SEQ_LEN, D_MODEL), jnp.float32)
    out_shapes = (
        enc_shape, enc_shape, enc_shape, enc_shape,
        jax.ShapeDtypeStruct((2, SEQ_LEN, SEQ_LEN), jnp.int8),
        jax.ShapeDtypeStruct((SEQ_LEN, SEQ_LEN), jnp.float32),
    )
    enc_spec = pl.BlockSpec((BLOCK_ROWS, D_MODEL), lambda i: (i, 0))
    out_specs = (
        enc_spec, enc_spec, enc_spec, enc_spec,
        pl.BlockSpec((2, BLOCK_ROWS, SEQ_LEN), lambda i: (0, i, 0)),
        pl.BlockSpec((BLOCK_ROWS, SEQ_LEN), lambda i: (i, 0)),
    )
    in_specs = [pl.BlockSpec((2, SEQ_LEN), lambda i: (0, 0))]
    return pl.pallas_call(
        _attn_struct_kernel,
        grid=(NUM_BLOCKS,),
        in_specs=in_specs,
        out_specs=out_specs,
        out_shape=out_shapes,
        scratch_shapes=[
            pltpu.VMEM((BLOCK_ROWS, D_HALF), jnp.float32),
            pltpu.VMEM((BLOCK_ROWS, D_HALF), jnp.float32),
        ],
        interpret=interpret,
    )(seg_id)


def kernel(hidden, seg_id, input_mask):
    del hidden
    q1, q2, k1, k2, seg_i8, func_mask = _run(seg_id)
    seg_mat = seg_i8.astype(jnp.bool_)
    attn_mask = input_mask[:, None, None, :]
    return (q1, q2, k1, k2, seg_mat, attn_mask, func_mask)
